# Initial kernel scaffold; baseline (speedup 1.0000x reference)
import jax
import jax.numpy as jnp
from jax.experimental import pallas as pl


def kernel(decoder_embed, all_cls_scores, all_bbox_preds, all_pts_preds, params):
    raise NotImplementedError("write your pallas kernel here")


# speedup vs baseline: 7.1234x; 7.1234x over previous
import functools

import jax
import jax.numpy as jnp
import numpy as np
from jax.experimental import pallas as pl

NB = 6
P = 20
NBP = NB * P
BIG = 1e30
NEG = -1e30


def _mm(a, b):
    return jax.lax.dot_general(a, b, (((1,), (0,)), ((), ())),
                               preferred_element_type=jnp.float32)


def _mmt(a, b):
    return jax.lax.dot_general(a, b, (((1,), (1,)), ((), ())),
                               preferred_element_type=jnp.float32)


def _knn_onehots(x, dmask):
    xsq = x * x
    ones = jnp.ones((1, x.shape[1]), jnp.float32)
    sq_row = _mmt(ones, xsq)
    xxt = _mmt(x, x)
    d = sq_row - 2.0 * xxt + dmask
    iota_q = jax.lax.broadcasted_iota(jnp.int32, (NBP, NBP), 1)
    onehots = []
    for _ in range(4):
        m = jnp.min(d, axis=1, keepdims=True)
        cand = jnp.where(d <= m, iota_q, NBP)
        sel = jnp.min(cand, axis=1, keepdims=True)
        oh = iota_q == sel
        onehots.append(oh.astype(jnp.float32))
        d = jnp.where(oh, BIG, d)
    return onehots


def _seg_extreme(v, rmod, combine, fill):
    for s in (1, 2, 4, 8, 16):
        shifted = jnp.concatenate(
            [v[s:, :], jnp.full((s, v.shape[1]), fill, jnp.float32)], axis=0)
        shifted = jnp.where(rmod + s < P, shifted, fill)
        v = combine(v, shifted)
    return v


def _block_kernel(pts_ref, emb_ref, cls_ref, dmask_ref, sel6_ref,
                  w0f_ref, w0r_ref, b0_ref, w1_ref, b1_ref, w2_ref, b2_ref,
                  gw1_ref, gb1_ref, gw2_ref, gb2_ref,
                  cw1_ref, cb1_ref, cw2_ref, cb2_ref, dw_ref,
                  cls_out_ref, coords_ref, out_ref):
    pts = pts_ref[...]
    dmask = dmask_ref[...]
    sel6 = sel6_ref[...]
    rmod = jax.lax.broadcasted_iota(jnp.int32, (NBP, 1), 0) % P

    shared_ohs = _knn_onehots(pts, dmask)

    dw = dw_ref[...]
    cw1 = cw1_ref[...]
    pre_cls = jnp.zeros((NB, 256), jnp.float32) + cb1_ref[...]
    down = jnp.zeros((NBP, 2), jnp.float32)

    def consume(piece, off, width):
        nonlocal pre_cls, down
        pm = _seg_extreme(piece, rmod, jnp.maximum, NEG)
        pooled = _mm(sel6, pm)
        pre_cls = pre_cls + _mm(
            pooled, jax.lax.slice_in_dim(cw1, off, off + width, axis=0))
        down = down + _mm(piece, jax.lax.slice_in_dim(dw, off, off + width, axis=0))

    for u in range(3):
        h = pts
        ohs = shared_ohs
        for c in range(4):
            j = u * 4 + c
            if c > 0:
                ohs = _knn_onehots(h, dmask)
            w0 = w0f_ref[u] if c == 0 else w0r_ref[u * 3 + c - 1]
            cin = h.shape[1]
            pre = _mm(h, w0[:cin]) + b0_ref[j]
            xb = _mm(h, w0[cin:])
            base = pre - xb
            acc = None
            for oh in ohs:
                g = jax.nn.relu(base + _mm(oh, xb))
                g = jax.nn.relu(_mm(g, w1_ref[j]) + b1_ref[j])
                g = jax.nn.relu(_mm(g, w2_ref[j]) + b2_ref[j])
                acc = g if acc is None else jnp.maximum(acc, g)
            h = acc
            consume(h, u * 96 + c * 24, 24)

    hg = jax.nn.relu(_mm(pts, gw1_ref[...]) + gb1_ref[...])
    amat = shared_ohs[0] + shared_ohs[1] + shared_ohs[2] + shared_ohs[3]
    agg = 0.5 * hg + 0.125 * _mm(amat, hg)
    g = jax.nn.relu(_mm(agg, gw2_ref[...]) + gb2_ref[...])
    consume(g, 288, 256)

    consume(emb_ref[...], 544, 256)

    logits = cls_ref[0] + _mm(jax.nn.relu(pre_cls), cw2_ref[...]) + cb2_ref[...]
    cls_out_ref[...] = logits[None]

    pos = pts + 0.05 * jnp.tanh(down)
    out_ref[...] = pos
    pmax = _mm(sel6, _seg_extreme(pos, rmod, jnp.maximum, NEG))
    pmin = _mm(sel6, _seg_extreme(pos, rmod, jnp.minimum, BIG))
    ctr = (pmin + pmax) * 0.5
    ext = pmax - pmin
    coords_ref[...] = jnp.concatenate(
        [ctr[:, 0:1], ctr[:, 1:2], ext[:, 0:1], ext[:, 1:2]], axis=1)[None]


@jax.jit
def _run(pts, emb, cls, dmask, sel6, weights):
    n = cls.shape[0] * NB
    grid = (n // NB,)

    def rows(width):
        return pl.BlockSpec((NBP, width), lambda i: (i, 0))

    def inst(width):
        return pl.BlockSpec((1, NB, width), lambda i: (i, 0, 0))

    def const(shape):
        nd = len(shape)
        return pl.BlockSpec(shape, lambda i: (0,) * nd)

    w_specs = [const(w.shape) for w in weights]
    out_shapes = (
        jax.ShapeDtypeStruct((n // NB, NB, 3), jnp.float32),
        jax.ShapeDtypeStruct((n // NB, NB, 4), jnp.float32),
        jax.ShapeDtypeStruct((n * P, 2), jnp.float32),
    )
    out_specs = (inst(3), inst(4), rows(2))
    return pl.pallas_call(
        _block_kernel,
        grid=grid,
        in_specs=[rows(2), rows(256), inst(3),
                  const((NBP, NBP)), const((NB, NBP))] + w_specs,
        out_specs=out_specs,
        out_shape=out_shapes,
    )(pts, emb, cls, dmask, sel6, *weights)


def _consts():
    dmask = np.full((NBP, NBP), BIG, np.float32)
    for j in range(NB):
        dmask[j * P:(j + 1) * P, j * P:(j + 1) * P] = 0.0
    dmask[np.arange(NBP), np.arange(NBP)] = 1e9
    sel6 = np.zeros((NB, NBP), np.float32)
    sel6[np.arange(NB), np.arange(NB) * P] = 1.0
    return jnp.asarray(dmask), jnp.asarray(sel6)


def kernel(decoder_embed, all_cls_scores, all_bbox_preds, all_pts_preds, params):
    L, B, S, P_, C = all_pts_preds.shape
    n = L * B * S
    pts = all_pts_preds.reshape(n * P_, C)
    cls = all_cls_scores.reshape(n // NB, NB, -1)
    emb = decoder_embed.reshape(n * P_, 256)

    edge = params['edge']
    w0f = jnp.stack([edge[u][0][0][0] for u in range(3)])
    w0r = jnp.stack([edge[u][c][0][0] for u in range(3) for c in range(1, 4)])
    b0 = jnp.stack([edge[u][c][0][1] for u in range(3) for c in range(4)])
    w1 = jnp.stack([edge[u][c][1][0] for u in range(3) for c in range(4)])
    b1 = jnp.stack([edge[u][c][1][1] for u in range(3) for c in range(4)])
    w2 = jnp.stack([edge[u][c][2][0] for u in range(3) for c in range(4)])
    b2 = jnp.stack([edge[u][c][2][1] for u in range(3) for c in range(4)])
    gp, cp, dp = params['gcn'], params['cls'], params['down']
    weights = [w0f, w0r, b0, w1, b1, w2, b2,
               gp['W1'], gp['b1'].reshape(1, -1), gp['W2'], gp['b2'].reshape(1, -1),
               cp['W1'], cp['b1'].reshape(1, -1), cp['W2'], cp['b2'].reshape(1, -1),
               dp['W']]
    dmask, sel6 = _consts()

    logits, coords, out = _run(pts, emb, cls, dmask, sel6, weights)
    return (logits.reshape(L, B, S, -1),
            coords.reshape(L, B, S, 4),
            out.reshape(L, B, S, P_, C))


# speedup vs baseline: 8.0342x; 1.1278x over previous
import jax
import jax.numpy as jnp
import numpy as np
from jax.experimental import pallas as pl

NB = 6
NSUB = 2
P = 20
NBP = NB * P
BIG = 1e30
NEG = -1e30


def _mm(a, b):
    return jax.lax.dot_general(a, b, (((1,), (0,)), ((), ())),
                               preferred_element_type=jnp.float32)


def _mmt(a, b):
    return jax.lax.dot_general(a, b, (((1,), (1,)), ((), ())),
                               preferred_element_type=jnp.float32)


def _knn_onehots(x, dmask):
    xsq = x * x
    ones = jnp.ones((1, x.shape[1]), jnp.float32)
    sq_row = _mmt(ones, xsq)
    xxt = _mmt(x, x)
    d = sq_row - 2.0 * xxt + dmask
    iota_q = jax.lax.broadcasted_iota(jnp.int32, (NBP, NBP), 1)
    onehots = []
    for _ in range(4):
        m = jnp.min(d, axis=1, keepdims=True)
        cand = jnp.where(d <= m, iota_q, NBP)
        sel = jnp.min(cand, axis=1, keepdims=True)
        oh = iota_q == sel
        onehots.append(oh.astype(jnp.float32))
        d = jnp.where(oh, BIG, d)
    return jnp.concatenate(onehots, axis=0), onehots


def _seg_extreme(v, rmod, combine, fill):
    for s in (1, 2, 4, 8, 16):
        shifted = jnp.concatenate(
            [v[s:, :], jnp.full((s, v.shape[1]), fill, jnp.float32)], axis=0)
        shifted = jnp.where(rmod + s < P, shifted, fill)
        v = combine(v, shifted)
    return v


def _edge_conv(h, oh_stack, w0, b0, w1, b1, w2, b2):
    cin = h.shape[1]
    pre = _mm(h, w0[:cin]) + b0
    xb = _mm(h, w0[cin:])
    base = pre - xb
    base4 = jnp.concatenate([base, base, base, base], axis=0)
    g = jax.nn.relu(base4 + _mm(oh_stack, xb))
    g = jax.nn.relu(_mm(g, w1) + b1)
    g = jax.nn.relu(_mm(g, w2) + b2)
    return jnp.maximum(jnp.maximum(g[0:NBP], g[NBP:2 * NBP]),
                       jnp.maximum(g[2 * NBP:3 * NBP], g[3 * NBP:]))


def _one_block(pts, emb, cls6, dmask, sel6, rmod, wrefs):
    (w0f_ref, w0r_ref, b0_ref, w1_ref, b1_ref, w2_ref, b2_ref,
     gw1_ref, gb1_ref, gw2_ref, gb2_ref,
     cw1_ref, cb1_ref, cw2_ref, cb2_ref, dw_ref) = wrefs

    shared_stack, shared_ohs = _knn_onehots(pts, dmask)
    dw = dw_ref[...]
    cw1 = cw1_ref[...]
    pre_cls = jnp.zeros((NB, 256), jnp.float32) + cb1_ref[...]
    down = jnp.zeros((NBP, 2), jnp.float32)

    def consume(piece, off, width):
        nonlocal pre_cls, down
        pm = _seg_extreme(piece, rmod, jnp.maximum, NEG)
        pooled = _mm(sel6, pm)
        pre_cls = pre_cls + _mm(
            pooled, jax.lax.slice_in_dim(cw1, off, off + width, axis=0))
        down = down + _mm(piece, jax.lax.slice_in_dim(dw, off, off + width, axis=0))

    for u in range(3):
        h = pts
        stack = shared_stack
        outs = []
        for c in range(4):
            j = u * 4 + c
            if c > 0:
                stack, _ = _knn_onehots(h, dmask)
            w0 = w0f_ref[u] if c == 0 else w0r_ref[u * 3 + c - 1]
            h = _edge_conv(h, stack, w0, b0_ref[j],
                           w1_ref[j], b1_ref[j], w2_ref[j], b2_ref[j])
            outs.append(h)
        consume(jnp.concatenate(outs, axis=1), u * 96, 96)

    hg = jax.nn.relu(_mm(pts, gw1_ref[...]) + gb1_ref[...])
    amat = shared_ohs[0] + shared_ohs[1] + shared_ohs[2] + shared_ohs[3]
    agg = 0.5 * hg + 0.125 * _mm(amat, hg)
    g = jax.nn.relu(_mm(agg, gw2_ref[...]) + gb2_ref[...])
    consume(g, 288, 256)
    consume(emb, 544, 256)

    logits = cls6 + _mm(jax.nn.relu(pre_cls), cw2_ref[...]) + cb2_ref[...]
    pos = pts + 0.05 * jnp.tanh(down)
    pmax = _mm(sel6, _seg_extreme(pos, rmod, jnp.maximum, NEG))
    pmin = _mm(sel6, _seg_extreme(pos, rmod, jnp.minimum, BIG))
    ctr = (pmin + pmax) * 0.5
    ext = pmax - pmin
    coords = jnp.concatenate(
        [ctr[:, 0:1], ctr[:, 1:2], ext[:, 0:1], ext[:, 1:2]], axis=1)
    return logits, coords, pos


def _block_kernel(pts_ref, emb_ref, cls_ref, dmask_ref, sel6_ref,
                  w0f_ref, w0r_ref, b0_ref, w1_ref, b1_ref, w2_ref, b2_ref,
                  gw1_ref, gb1_ref, gw2_ref, gb2_ref,
                  cw1_ref, cb1_ref, cw2_ref, cb2_ref, dw_ref,
                  cls_out_ref, coords_ref, out_ref):
    dmask = dmask_ref[...]
    sel6 = sel6_ref[...]
    rmod = jax.lax.broadcasted_iota(jnp.int32, (NBP, 1), 0) % P
    wrefs = (w0f_ref, w0r_ref, b0_ref, w1_ref, b1_ref, w2_ref, b2_ref,
             gw1_ref, gb1_ref, gw2_ref, gb2_ref,
             cw1_ref, cb1_ref, cw2_ref, cb2_ref, dw_ref)
    for s in range(NSUB):
        r0 = s * NBP
        logits, coords, pos = _one_block(
            pts_ref[r0:r0 + NBP, :], emb_ref[r0:r0 + NBP, :], cls_ref[s],
            dmask, sel6, rmod, wrefs)
        cls_out_ref[s] = logits
        coords_ref[s] = coords
        out_ref[r0:r0 + NBP, :] = pos


@jax.jit
def _run(pts, emb, cls, dmask, sel6, weights):
    n = cls.shape[0] * NB
    grid = (n // (NB * NSUB),)

    def rows(width):
        return pl.BlockSpec((NSUB * NBP, width), lambda i: (i, 0))

    def inst(width):
        return pl.BlockSpec((NSUB, NB, width), lambda i: (i, 0, 0))

    def const(shape):
        nd = len(shape)
        return pl.BlockSpec(shape, lambda i: (0,) * nd)

    w_specs = [const(w.shape) for w in weights]
    out_shapes = (
        jax.ShapeDtypeStruct((n // NB, NB, 3), jnp.float32),
        jax.ShapeDtypeStruct((n // NB, NB, 4), jnp.float32),
        jax.ShapeDtypeStruct((n * P, 2), jnp.float32),
    )
    out_specs = (inst(3), inst(4), rows(2))
    return pl.pallas_call(
        _block_kernel,
        grid=grid,
        in_specs=[rows(2), rows(256), inst(3),
                  const((NBP, NBP)), const((NB, NBP))] + w_specs,
        out_specs=out_specs,
        out_shape=out_shapes,
    )(pts, emb, cls, dmask, sel6, *weights)


def _consts():
    dmask = np.full((NBP, NBP), BIG, np.float32)
    for j in range(NB):
        dmask[j * P:(j + 1) * P, j * P:(j + 1) * P] = 0.0
    dmask[np.arange(NBP), np.arange(NBP)] = 1e9
    sel6 = np.zeros((NB, NBP), np.float32)
    sel6[np.arange(NB), np.arange(NB) * P] = 1.0
    return jnp.asarray(dmask), jnp.asarray(sel6)


def kernel(decoder_embed, all_cls_scores, all_bbox_preds, all_pts_preds, params):
    L, B, S, P_, C = all_pts_preds.shape
    n = L * B * S
    pts = all_pts_preds.reshape(n * P_, C)
    cls = all_cls_scores.reshape(n // NB, NB, -1)
    emb = decoder_embed.reshape(n * P_, 256)

    edge = params['edge']
    w0f = jnp.stack([edge[u][0][0][0] for u in range(3)])
    w0r = jnp.stack([edge[u][c][0][0] for u in range(3) for c in range(1, 4)])
    b0 = jnp.stack([edge[u][c][0][1] for u in range(3) for c in range(4)])
    w1 = jnp.stack([edge[u][c][1][0] for u in range(3) for c in range(4)])
    b1 = jnp.stack([edge[u][c][1][1] for u in range(3) for c in range(4)])
    w2 = jnp.stack([edge[u][c][2][0] for u in range(3) for c in range(4)])
    b2 = jnp.stack([edge[u][c][2][1] for u in range(3) for c in range(4)])
    gp, cp, dp = params['gcn'], params['cls'], params['down']
    weights = [w0f, w0r, b0, w1, b1, w2, b2,
               gp['W1'], gp['b1'].reshape(1, -1), gp['W2'], gp['b2'].reshape(1, -1),
               cp['W1'], cp['b1'].reshape(1, -1), cp['W2'], cp['b2'].reshape(1, -1),
               dp['W']]
    dmask, sel6 = _consts()

    logits, coords, out = _run(pts, emb, cls, dmask, sel6, weights)
    return (logits.reshape(L, B, S, -1),
            coords.reshape(L, B, S, 4),
            out.reshape(L, B, S, P_, C))


# speedup vs baseline: 13.4922x; 1.6794x over previous
import jax
import jax.numpy as jnp
import numpy as np
from jax.experimental import pallas as pl

NB = 6
NSUB = 2
P = 20
NBP = NB * P
BIG = 1e30
NEG = -1e30


def _mm(a, b):
    return jax.lax.dot_general(a, b, (((1,), (0,)), ((), ())),
                               preferred_element_type=jnp.float32)


def _mmt(a, b):
    return jax.lax.dot_general(a, b, (((1,), (1,)), ((), ())),
                               preferred_element_type=jnp.float32)


def _mtm(a, b):
    return jax.lax.dot_general(a, b, (((0,), (0,)), ((), ())),
                               preferred_element_type=jnp.float32)


def _knn_onehots(x, dmask, bdmask):
    xsq = x * x
    onesc = jnp.ones((x.shape[1], 1), jnp.float32)
    sq_col = _mm(xsq, onesc)
    xxt = _mmt(x, x)
    d = sq_col - 2.0 * xxt + dmask
    dc = d[0:P]
    for n in range(1, NB):
        dc = jnp.minimum(dc, d[n * P:(n + 1) * P])
    iota_r = jax.lax.broadcasted_iota(jnp.int32, (P, NBP), 0).astype(jnp.float32)
    onehots = []
    for _ in range(4):
        m = jnp.min(dc, axis=0, keepdims=True)
        cand = jnp.where(dc <= m, iota_r, float(P))
        sel = jnp.min(cand, axis=0, keepdims=True)
        ohb = iota_r == sel
        ohc = ohb.astype(jnp.float32)
        ohf = jnp.concatenate([ohc] * NB, axis=0) * bdmask
        onehots.append(ohf)
        dc = jnp.where(ohb, BIG, dc)
    return onehots


def _seg_extreme(v, rmod, combine, fill):
    for s in (1, 2, 4, 8, 16):
        shifted = jnp.concatenate(
            [v[s:, :], jnp.full((s, v.shape[1]), fill, jnp.float32)], axis=0)
        shifted = jnp.where(rmod + s < P, shifted, fill)
        v = combine(v, shifted)
    return v


def _edge_conv(h, ohs, w0, b0, w1, b1, w2, b2):
    cin = h.shape[1]
    pre = _mm(h, w0[:cin]) + b0
    xb = _mm(h, w0[cin:])
    base = pre - xb
    base4 = jnp.concatenate([base, base, base, base], axis=0)
    gat = jnp.concatenate([_mtm(o, xb) for o in ohs], axis=0)
    g = jax.nn.relu(base4 + gat)
    g = jax.nn.relu(_mm(g, w1) + b1)
    g = jax.nn.relu(_mm(g, w2) + b2)
    return jnp.maximum(jnp.maximum(g[0:NBP], g[NBP:2 * NBP]),
                       jnp.maximum(g[2 * NBP:3 * NBP], g[3 * NBP:]))


def _one_block(pts, emb, cls6, dmask, bdmask, sel6, rmod, wrefs):
    (w0f_ref, w0r_ref, b0_ref, w1_ref, b1_ref, w2_ref, b2_ref,
     gw1_ref, gb1_ref, gw2_ref, gb2_ref,
     cw1_ref, cb1_ref, cw2_ref, cb2_ref, dw_ref) = wrefs

    shared_ohs = _knn_onehots(pts, dmask, bdmask)
    dw = dw_ref[...]
    cw1 = cw1_ref[...]
    pre_cls = jnp.zeros((NB, 256), jnp.float32) + cb1_ref[...]
    down = jnp.zeros((NBP, 2), jnp.float32)

    def consume(piece, off, width):
        nonlocal pre_cls, down
        pm = _seg_extreme(piece, rmod, jnp.maximum, NEG)
        pooled = _mm(sel6, pm)
        pre_cls = pre_cls + _mm(
            pooled, jax.lax.slice_in_dim(cw1, off, off + width, axis=0))
        down = down + _mm(piece, jax.lax.slice_in_dim(dw, off, off + width, axis=0))

    for u in range(3):
        h = pts
        ohs = shared_ohs
        outs = []
        for c in range(4):
            j = u * 4 + c
            if c > 0:
                ohs = _knn_onehots(h, dmask, bdmask)
            w0 = w0f_ref[u] if c == 0 else w0r_ref[u * 3 + c - 1]
            h = _edge_conv(h, ohs, w0, b0_ref[j],
                           w1_ref[j], b1_ref[j], w2_ref[j], b2_ref[j])
            outs.append(h)
        consume(jnp.concatenate(outs, axis=1), u * 96, 96)

    hg = jax.nn.relu(_mm(pts, gw1_ref[...]) + gb1_ref[...])
    amat = shared_ohs[0] + shared_ohs[1] + shared_ohs[2] + shared_ohs[3]
    agg = 0.5 * hg + 0.125 * _mtm(amat, hg)
    g = jax.nn.relu(_mm(agg, gw2_ref[...]) + gb2_ref[...])
    consume(g, 288, 256)
    consume(emb, 544, 256)

    logits = cls6 + _mm(jax.nn.relu(pre_cls), cw2_ref[...]) + cb2_ref[...]
    pos = pts + 0.05 * jnp.tanh(down)
    pmax = _mm(sel6, _seg_extreme(pos, rmod, jnp.maximum, NEG))
    pmin = _mm(sel6, _seg_extreme(pos, rmod, jnp.minimum, BIG))
    ctr = (pmin + pmax) * 0.5
    ext = pmax - pmin
    coords = jnp.concatenate(
        [ctr[:, 0:1], ctr[:, 1:2], ext[:, 0:1], ext[:, 1:2]], axis=1)
    return logits, coords, pos


def _block_kernel(pts_ref, emb_ref, cls_ref, dmask_ref, bdmask_ref, sel6_ref,
                  w0f_ref, w0r_ref, b0_ref, w1_ref, b1_ref, w2_ref, b2_ref,
                  gw1_ref, gb1_ref, gw2_ref, gb2_ref,
                  cw1_ref, cb1_ref, cw2_ref, cb2_ref, dw_ref,
                  cls_out_ref, coords_ref, out_ref):
    dmask = dmask_ref[...]
    bdmask = bdmask_ref[...]
    sel6 = sel6_ref[...]
    rmod = jax.lax.broadcasted_iota(jnp.int32, (NBP, 1), 0) % P
    wrefs = (w0f_ref, w0r_ref, b0_ref, w1_ref, b1_ref, w2_ref, b2_ref,
             gw1_ref, gb1_ref, gw2_ref, gb2_ref,
             cw1_ref, cb1_ref, cw2_ref, cb2_ref, dw_ref)
    for s in range(NSUB):
        r0 = s * NBP
        logits, coords, pos = _one_block(
            pts_ref[r0:r0 + NBP, :], emb_ref[r0:r0 + NBP, :], cls_ref[s],
            dmask, bdmask, sel6, rmod, wrefs)
        cls_out_ref[s] = logits
        coords_ref[s] = coords
        out_ref[r0:r0 + NBP, :] = pos


@jax.jit
def _run(pts, emb, cls, dmask, bdmask, sel6, weights):
    n = cls.shape[0] * NB
    grid = (n // (NB * NSUB),)

    def rows(width):
        return pl.BlockSpec((NSUB * NBP, width), lambda i: (i, 0))

    def inst(width):
        return pl.BlockSpec((NSUB, NB, width), lambda i: (i, 0, 0))

    def const(shape):
        nd = len(shape)
        return pl.BlockSpec(shape, lambda i: (0,) * nd)

    w_specs = [const(w.shape) for w in weights]
    out_shapes = (
        jax.ShapeDtypeStruct((n // NB, NB, 3), jnp.float32),
        jax.ShapeDtypeStruct((n // NB, NB, 4), jnp.float32),
        jax.ShapeDtypeStruct((n * P, 2), jnp.float32),
    )
    out_specs = (inst(3), inst(4), rows(2))
    return pl.pallas_call(
        _block_kernel,
        grid=grid,
        in_specs=[rows(2), rows(256), inst(3),
                  const((NBP, NBP)), const((NBP, NBP)), const((NB, NBP))] + w_specs,
        out_specs=out_specs,
        out_shape=out_shapes,
    )(pts, emb, cls, dmask, bdmask, sel6, *weights)


def _consts():
    dmask = np.full((NBP, NBP), BIG, np.float32)
    bdmask = np.zeros((NBP, NBP), np.float32)
    for j in range(NB):
        dmask[j * P:(j + 1) * P, j * P:(j + 1) * P] = 0.0
        bdmask[j * P:(j + 1) * P, j * P:(j + 1) * P] = 1.0
    dmask[np.arange(NBP), np.arange(NBP)] = 1e9
    sel6 = np.zeros((NB, NBP), np.float32)
    sel6[np.arange(NB), np.arange(NB) * P] = 1.0
    return jnp.asarray(dmask), jnp.asarray(bdmask), jnp.asarray(sel6)


def kernel(decoder_embed, all_cls_scores, all_bbox_preds, all_pts_preds, params):
    L, B, S, P_, C = all_pts_preds.shape
    n = L * B * S
    pts = all_pts_preds.reshape(n * P_, C)
    cls = all_cls_scores.reshape(n // NB, NB, -1)
    emb = decoder_embed.reshape(n * P_, 256)

    edge = params['edge']
    w0f = jnp.stack([edge[u][0][0][0] for u in range(3)])
    w0r = jnp.stack([edge[u][c][0][0] for u in range(3) for c in range(1, 4)])
    b0 = jnp.stack([edge[u][c][0][1] for u in range(3) for c in range(4)])
    w1 = jnp.stack([edge[u][c][1][0] for u in range(3) for c in range(4)])
    b1 = jnp.stack([edge[u][c][1][1] for u in range(3) for c in range(4)])
    w2 = jnp.stack([edge[u][c][2][0] for u in range(3) for c in range(4)])
    b2 = jnp.stack([edge[u][c][2][1] for u in range(3) for c in range(4)])
    gp, cp, dp = params['gcn'], params['cls'], params['down']
    weights = [w0f, w0r, b0, w1, b1, w2, b2,
               gp['W1'], gp['b1'].reshape(1, -1), gp['W2'], gp['b2'].reshape(1, -1),
               cp['W1'], cp['b1'].reshape(1, -1), cp['W2'], cp['b2'].reshape(1, -1),
               dp['W']]
    dmask, bdmask, sel6 = _consts()

    logits, coords, out = _run(pts, emb, cls, dmask, bdmask, sel6, weights)
    return (logits.reshape(L, B, S, -1),
            coords.reshape(L, B, S, 4),
            out.reshape(L, B, S, P_, C))
